# baseline (speedup 1.0000x reference)
import functools

import jax
import jax.numpy as jnp
import numpy as np
from jax import lax
from jax.experimental import pallas as pl
from jax.experimental.pallas import tpu as pltpu
from jax.experimental.pallas import tpu_sc as plsc

_N = 20000
_NC, _NS = 2, 16
_NW = _NC * _NS
_LW = 1280

_HALF_PI = np.float32(np.pi / 2.0)


def _broadcast_lane(buf, i):
    return plsc.load_gather(buf, [jnp.full((16,), i, dtype=jnp.int32)])


_mesh = plsc.VectorSubcoreMesh(core_axis_name="c", subcore_axis_name="s", num_cores=1)


@functools.partial(
    pl.kernel,
    out_type=[
        jax.ShapeDtypeStruct((7, _N), jnp.float32),
        jax.ShapeDtypeStruct((4, _N), jnp.float32),
        jax.ShapeDtypeStruct((7, _N), jnp.float32),
        jax.ShapeDtypeStruct((4, _N), jnp.float32),
    ],
    mesh=_mesh,
    compiler_params=pltpu.CompilerParams(needs_layout_passes=False),
    scratch_types=[
        pltpu.VMEM((16,), jnp.float32),
        pltpu.VMEM((16,), jnp.float32),
        pltpu.VMEM((7, 2 * _LW), jnp.float32),
        pltpu.VMEM((4, 2 * _LW), jnp.float32),
        pltpu.VMEM((2 * _LW,), jnp.float32),
        pltpu.VMEM((2, 2 * _LW), jnp.float32),
        pltpu.VMEM((7, 2 * _LW), jnp.float32),
        pltpu.VMEM((4, 2 * _LW), jnp.float32),
        pltpu.SemaphoreType.DMA,
        pltpu.SemaphoreType.DMA,
    ],
)
def _sc_fwd(pb1, ps1, po1, g1, pb2, ps2, po2, g2, orients, dims,
            ob1, os1, ob2, os2,
            c_vm, d_vm, pb_v, ps_v, po_v, g_v, tb_v, so_v,
            semi, semo):
    wid = lax.axis_index("s") * _NC + lax.axis_index("c")
    l0 = 128 * jnp.minimum(10 * wid, 147)

    heads = ((pb1, ps1, po1, g1, ob1, os1), (pb2, ps2, po2, g2, ob2, os2))
    in_cps = []
    for h, (pbh, psh, poh, gh, _, _) in enumerate(heads):
        o = h * _LW
        in_cps += [
            pltpu.async_copy(pbh.at[:, pl.ds(l0, _LW)], pb_v.at[:, pl.ds(o, _LW)], semi),
            pltpu.async_copy(psh.at[:, pl.ds(l0, _LW)], ps_v.at[:, pl.ds(o, _LW)], semi),
            pltpu.async_copy(poh.at[pl.ds(l0, _LW)], po_v.at[pl.ds(o, _LW)], semi),
            pltpu.async_copy(gh.at[:, pl.ds(l0, _LW)], g_v.at[:, pl.ds(o, _LW)], semi),
        ]

    pltpu.sync_copy(orients, c_vm.at[pl.ds(8, 2)])
    pltpu.sync_copy(dims, d_vm.at[pl.ds(8, 3)])
    d0 = _broadcast_lane(d_vm, 8)
    d1 = _broadcast_lane(d_vm, 9)
    d2 = _broadcast_lane(d_vm, 10)
    o0 = _broadcast_lane(c_vm, 8)
    o1 = _broadcast_lane(c_vm, 9)

    for cp in in_cps:
        cp.wait()

    @plsc.parallel_loop(0, 2 * _LW, step=16, unroll=2)
    def body(s):
        orient_v = jnp.where(s < _LW, o0, o1)
        po_l = po_v[pl.ds(s, 16)]
        mk = jnp.where(po_l >= 0.9, 1.0, 0.0).astype(jnp.float32)
        x0 = pb_v[0, pl.ds(s, 16)]
        s0 = 1.0 / (1.0 + jnp.exp(-x0))
        tb_v[0, pl.ds(s, 16)] = (orient_v + s0 * _HALF_PI) * mk
        for c in (1, 2):
            x = pb_v[c, pl.ds(s, 16)]
            gv = g_v[c - 1, pl.ds(s, 16)]
            sg = 1.0 / (1.0 + jnp.exp(-x))
            tb_v[c, pl.ds(s, 16)] = (sg + gv + 0.5) * mk
        x3 = pb_v[3, pl.ds(s, 16)]
        s3 = 1.0 / (1.0 + jnp.exp(-x3))
        tb_v[3, pl.ds(s, 16)] = s3 * mk
        for c, dv in ((4, d0), (5, d1), (6, d2)):
            x = pb_v[c, pl.ds(s, 16)]
            tb_v[c, pl.ds(s, 16)] = dv * jnp.exp(x) * mk
        for c in range(4):
            so_v[c, pl.ds(s, 16)] = ps_v[c, pl.ds(s, 16)] * mk

    out_cps = []
    for h, (_, _, _, _, obh, osh) in enumerate(heads):
        o = h * _LW
        out_cps += [
            pltpu.async_copy(tb_v.at[:, pl.ds(o, _LW)], obh.at[:, pl.ds(l0, _LW)], semo),
            pltpu.async_copy(so_v.at[:, pl.ds(o, _LW)], osh.at[:, pl.ds(l0, _LW)], semo),
        ]
    for cp in out_cps:
        cp.wait()


def kernel(pred_bboxes1, pred_class_scores1, pred_objectness1, pred_bboxes_grid1,
           pred_bboxes2, pred_class_scores2, pred_objectness2, pred_bboxes_grid2,
           anchor_orients, anchor_dims):
    ob1, os1, ob2, os2 = _sc_fwd(
        pred_bboxes1.T, pred_class_scores1.T, pred_objectness1, pred_bboxes_grid1.T,
        pred_bboxes2.T, pred_class_scores2.T, pred_objectness2, pred_bboxes_grid2.T,
        anchor_orients, anchor_dims,
    )
    return (ob1.T, os1.T, ob2.T, os2.T)


# speedup vs baseline: 1.0451x; 1.0451x over previous
import functools

import jax
import jax.numpy as jnp
import numpy as np
from jax import lax
from jax.experimental import pallas as pl
from jax.experimental.pallas import tpu as pltpu
from jax.experimental.pallas import tpu_sc as plsc

_N = 20000
_NC, _NS = 2, 16
_NW = _NC * _NS
_LW = 640

_HALF_PI = np.float32(np.pi / 2.0)


def _broadcast_lane(buf, i):
    return plsc.load_gather(buf, [jnp.full((16,), i, dtype=jnp.int32)])


_mesh = plsc.VectorSubcoreMesh(core_axis_name="c", subcore_axis_name="s")


@functools.partial(
    pl.kernel,
    out_type=[
        jax.ShapeDtypeStruct((7, _N), jnp.float32),
        jax.ShapeDtypeStruct((4, _N), jnp.float32),
        jax.ShapeDtypeStruct((7, _N), jnp.float32),
        jax.ShapeDtypeStruct((4, _N), jnp.float32),
    ],
    mesh=_mesh,
    compiler_params=pltpu.CompilerParams(needs_layout_passes=False),
    scratch_types=[
        pltpu.VMEM((16,), jnp.float32),
        pltpu.VMEM((16,), jnp.float32),
        pltpu.VMEM((7, 2 * _LW), jnp.float32),
        pltpu.VMEM((4, 2 * _LW), jnp.float32),
        pltpu.VMEM((2 * _LW,), jnp.float32),
        pltpu.VMEM((2, 2 * _LW), jnp.float32),
        pltpu.VMEM((7, 2 * _LW), jnp.float32),
        pltpu.VMEM((4, 2 * _LW), jnp.float32),
        pltpu.SemaphoreType.DMA,
        pltpu.SemaphoreType.DMA,
    ],
)
def _sc_fwd(pb1, ps1, po1, g1, pb2, ps2, po2, g2, orients, dims,
            ob1, os1, ob2, os2,
            c_vm, d_vm, pb_v, ps_v, po_v, g_v, tb_v, so_v,
            semi, semo):
    wid = lax.axis_index("s") * _NC + lax.axis_index("c")
    l0 = 128 * jnp.minimum(5 * wid, 152)

    heads = ((pb1, ps1, po1, g1, ob1, os1), (pb2, ps2, po2, g2, ob2, os2))
    in_cps = []
    for h, (pbh, psh, poh, gh, _, _) in enumerate(heads):
        o = h * _LW
        in_cps += [
            pltpu.async_copy(pbh.at[:, pl.ds(l0, _LW)], pb_v.at[:, pl.ds(o, _LW)], semi),
            pltpu.async_copy(psh.at[:, pl.ds(l0, _LW)], ps_v.at[:, pl.ds(o, _LW)], semi),
            pltpu.async_copy(poh.at[pl.ds(l0, _LW)], po_v.at[pl.ds(o, _LW)], semi),
            pltpu.async_copy(gh.at[:, pl.ds(l0, _LW)], g_v.at[:, pl.ds(o, _LW)], semi),
        ]

    pltpu.sync_copy(orients, c_vm.at[pl.ds(8, 2)])
    pltpu.sync_copy(dims, d_vm.at[pl.ds(8, 3)])
    d0 = _broadcast_lane(d_vm, 8)
    d1 = _broadcast_lane(d_vm, 9)
    d2 = _broadcast_lane(d_vm, 10)
    o0 = _broadcast_lane(c_vm, 8)
    o1 = _broadcast_lane(c_vm, 9)

    for cp in in_cps:
        cp.wait()

    @plsc.parallel_loop(0, 2 * _LW, step=16, unroll=2)
    def body(s):
        orient_v = jnp.where(s < _LW, o0, o1)
        po_l = po_v[pl.ds(s, 16)]
        mk = jnp.where(po_l >= 0.9, 1.0, 0.0).astype(jnp.float32)
        x0 = pb_v[0, pl.ds(s, 16)]
        s0 = 1.0 / (1.0 + jnp.exp(-x0))
        tb_v[0, pl.ds(s, 16)] = (orient_v + s0 * _HALF_PI) * mk
        for c in (1, 2):
            x = pb_v[c, pl.ds(s, 16)]
            gv = g_v[c - 1, pl.ds(s, 16)]
            sg = 1.0 / (1.0 + jnp.exp(-x))
            tb_v[c, pl.ds(s, 16)] = (sg + gv + 0.5) * mk
        x3 = pb_v[3, pl.ds(s, 16)]
        s3 = 1.0 / (1.0 + jnp.exp(-x3))
        tb_v[3, pl.ds(s, 16)] = s3 * mk
        for c, dv in ((4, d0), (5, d1), (6, d2)):
            x = pb_v[c, pl.ds(s, 16)]
            tb_v[c, pl.ds(s, 16)] = dv * jnp.exp(x) * mk
        for c in range(4):
            so_v[c, pl.ds(s, 16)] = ps_v[c, pl.ds(s, 16)] * mk

    out_cps = []
    for h, (_, _, _, _, obh, osh) in enumerate(heads):
        o = h * _LW
        out_cps += [
            pltpu.async_copy(tb_v.at[:, pl.ds(o, _LW)], obh.at[:, pl.ds(l0, _LW)], semo),
            pltpu.async_copy(so_v.at[:, pl.ds(o, _LW)], osh.at[:, pl.ds(l0, _LW)], semo),
        ]
    for cp in out_cps:
        cp.wait()


def kernel(pred_bboxes1, pred_class_scores1, pred_objectness1, pred_bboxes_grid1,
           pred_bboxes2, pred_class_scores2, pred_objectness2, pred_bboxes_grid2,
           anchor_orients, anchor_dims):
    ob1, os1, ob2, os2 = _sc_fwd(
        pred_bboxes1.T, pred_class_scores1.T, pred_objectness1, pred_bboxes_grid1.T,
        pred_bboxes2.T, pred_class_scores2.T, pred_objectness2, pred_bboxes_grid2.T,
        anchor_orients, anchor_dims,
    )
    return (ob1.T, os1.T, ob2.T, os2.T)
